# baseline (speedup 1.0000x reference)
import functools

import jax
import jax.numpy as jnp
from jax.experimental import pallas as pl
from jax.experimental.pallas import tpu as pltpu
from jax.experimental.pallas import tpu_sc as plsc

B, S, D, M, E, K = 4, 2048, 768, 3072, 8, 2
N = B * S
T = 512
NT = 40
NPAD = NT * T
NC, NS, L = 2, 16, 16
LB = 2 * L
NW = NC * NS
TOK_W = N // NW


def _rnd_bf16(v):
    lsb = jax.lax.shift_right_logical(v, 16) & 1
    return jax.lax.shift_right_logical(v + 0x7FFF + lsb, 16)


def _pack_bf16(yf):
    u = jax.lax.bitcast_convert_type(yf, jnp.int32)
    lo = _rnd_bf16(u[:, :D // 2])
    hi = _rnd_bf16(u[:, D // 2:])
    return lo | jax.lax.shift_left(hi, 16)


def _unpack_bf16(ui):
    lo = jax.lax.bitcast_convert_type(jax.lax.shift_left(ui, 16), jnp.float32)
    hi16 = jax.lax.shift_left(jax.lax.shift_right_logical(ui, 16), 16)
    hi = jax.lax.bitcast_convert_type(hi16, jnp.float32)
    return jnp.concatenate([lo, hi], axis=1)


def _cumsum0(a, n):
    k = 1
    while k < n:
        z = jnp.zeros((k,) + a.shape[1:], a.dtype)
        a = a + jnp.concatenate([z, a[:-k]], axis=0)
        k *= 2
    return a


_RB = 1024


def _router_body(x_ref, wr_ref, pr0_ref, pr1_ref, idx_ref, xbf_ref):
    x = x_ref[...]
    wr = wr_ref[...]
    xbf_ref[...] = _pack_bf16(x)
    logits = jax.lax.dot_general(x, wr, (((1,), (1,)), ((), ())),
                                 preferred_element_type=jnp.float32)
    m = jnp.max(logits, axis=-1, keepdims=True)
    p = jnp.exp(logits - m)
    probs = p / jnp.sum(p, axis=-1, keepdims=True)
    iota = jax.lax.broadcasted_iota(jnp.int32, (_RB, E), 1)
    m1 = jnp.max(probs, axis=-1, keepdims=True)
    i1 = jnp.min(jnp.where(probs == m1, iota, E), axis=-1, keepdims=True)
    probs2 = jnp.where(iota == i1, -1.0, probs)
    m2 = jnp.max(probs2, axis=-1, keepdims=True)
    i2 = jnp.min(jnp.where(probs2 == m2, iota, E), axis=-1, keepdims=True)
    ps = m1 + m2
    ones = jnp.ones((1, L), jnp.float32)
    pr0_ref[...] = (m1 / ps) * ones
    pr1_ref[...] = (m2 / ps) * ones
    idx_ref[...] = jnp.concatenate([i1, i2], axis=1)


def _router(xf, Wr):
    return pl.pallas_call(
        _router_body,
        grid=(N // _RB,),
        in_specs=[
            pl.BlockSpec((_RB, D), lambda i: (i, 0)),
            pl.BlockSpec((E, D), lambda i: (0, 0)),
        ],
        out_specs=(
            pl.BlockSpec((_RB, L), lambda i: (i, 0)),
            pl.BlockSpec((_RB, L), lambda i: (i, 0)),
            pl.BlockSpec((_RB, K), lambda i: (i, 0)),
            pl.BlockSpec((_RB, D // 2), lambda i: (i, 0)),
        ),
        out_shape=(
            jax.ShapeDtypeStruct((N, L), jnp.float32),
            jax.ShapeDtypeStruct((N, L), jnp.float32),
            jax.ShapeDtypeStruct((N, K), jnp.int32),
            jax.ShapeDtypeStruct((N, D // 2), jnp.int32),
        ),
    )(xf, Wr)


def _meta_body(idx_ref, dest_ref, te_ref):
    idx = idx_ref[...]
    iota = jax.lax.broadcasted_iota(jnp.int32, (N, E), 1)
    oh0 = (iota == idx[:, 0:1]).astype(jnp.float32)
    oh1 = (iota == idx[:, 1:2]).astype(jnp.float32)
    both = oh0 + oh1
    cum = _cumsum0(both, N)
    counts = cum[N - 1:N, :]
    excl = cum - both
    tiles_e = jnp.floor((counts + (T - 1)) * (1.0 / T))
    cumt = tiles_e
    k = 1
    while k < E:
        z = jnp.zeros((1, k), jnp.float32)
        cumt = cumt + jnp.concatenate([z, cumt[:, :-k]], axis=1)
        k *= 2
    off_pad = (cumt - tiles_e) * T
    base = off_pad + excl
    d0 = jnp.sum(base * oh0, axis=-1, keepdims=True)
    d1 = jnp.sum(base * oh1, axis=-1, keepdims=True)
    dest_ref[...] = jnp.concatenate([d0, d1], axis=1).astype(jnp.int32)

    j = jax.lax.broadcasted_iota(jnp.int32, (128, E), 0).astype(jnp.float32)
    te = jnp.sum((j >= cumt).astype(jnp.float32), axis=-1, keepdims=True)
    te = jnp.minimum(te, E - 1).astype(jnp.int32)
    te_ref[...] = te.reshape(1, 128)


def _meta(idx):
    return pl.pallas_call(
        _meta_body,
        out_shape=(
            jax.ShapeDtypeStruct((N, K), jnp.int32),
            jax.ShapeDtypeStruct((1, 128), jnp.int32),
        ),
    )(idx)


def _mlp_body(te_ref, x_ref, w1_ref, w2_ref, b1_ref, b2_ref, o_ref):
    i = pl.program_id(0)
    e = te_ref[i]
    xb = _unpack_bf16(x_ref[...])
    w1 = w1_ref[0]
    h = jax.lax.dot_general(xb, w1, (((1,), (1,)), ((), ())),
                            preferred_element_type=jnp.float32)
    h = h + b1_ref[pl.ds(e, 1), :]
    h = 0.5 * h * (1.0 + jax.lax.erf(h * 0.7071067811865476))
    w2 = w2_ref[0]
    y = jax.lax.dot_general(h, w2, (((1,), (1,)), ((), ())),
                            preferred_element_type=jnp.float32)
    o_ref[...] = y + b2_ref[pl.ds(e, 1), :]


def _grouped_mlp(x_sorted, fc1_w, fc1_b, fc2_w, fc2_b, te):
    grid_spec = pltpu.PrefetchScalarGridSpec(
        num_scalar_prefetch=1,
        grid=(NT,),
        in_specs=[
            pl.BlockSpec((T, D // 2), lambda i, te_ref: (i, 0)),
            pl.BlockSpec((1, M, D), lambda i, te_ref: (te_ref[i], 0, 0)),
            pl.BlockSpec((1, D, M), lambda i, te_ref: (te_ref[i], 0, 0)),
            pl.BlockSpec((E, M), lambda i, te_ref: (0, 0)),
            pl.BlockSpec((E, D), lambda i, te_ref: (0, 0)),
        ],
        out_specs=pl.BlockSpec((T, D), lambda i, te_ref: (i, 0)),
    )
    return pl.pallas_call(
        _mlp_body,
        grid_spec=grid_spec,
        out_shape=jax.ShapeDtypeStruct((NPAD, D), jnp.float32),
    )(te, x_sorted, fc1_w, fc2_w, fc1_b, fc2_b)


_SC_MESH = plsc.VectorSubcoreMesh(core_axis_name="c", subcore_axis_name="s")
_CD = 128
_CC = 32


@functools.partial(
    pl.kernel, mesh=_SC_MESH,
    out_type=jax.ShapeDtypeStruct((NPAD, D // 2), jnp.int32),
    scratch_types=[
        pltpu.VMEM((_CD, D // 2), jnp.int32),
        pltpu.VMEM((_CD, D // 2), jnp.int32),
        pltpu.VMEM((_CD,), jnp.int32),
        pltpu.VMEM((_CD,), jnp.int32),
        pltpu.VMEM((_CD,), jnp.int32),
        pltpu.VMEM((_CD,), jnp.int32),
        pltpu.SemaphoreType.DMA,
        pltpu.SemaphoreType.DMA,
    ],
)
def _dispatch(x_hbm, d0_hbm, d1_hbm, xs_hbm,
              xba, xbb, i0a, i1a, i0b, i1b, sema, semb):
    wid = jax.lax.axis_index("s") * NC + jax.lax.axis_index("c")
    sets = ((xba, i0a, i1a, sema), (xbb, i0b, i1b, semb))

    for c in range(TOK_W // _CD):
        xb, i0, i1, sm = sets[c % 2]
        base = wid * TOK_W + c * _CD
        pltpu.sync_copy(x_hbm.at[pl.ds(base, _CD), :], xb)
        pltpu.sync_copy(d0_hbm.at[pl.ds(base, _CD)], i0)
        pltpu.sync_copy(d1_hbm.at[pl.ds(base, _CD)], i1)
        pltpu.async_copy(xb, xs_hbm.at[i0], sm)
        pltpu.async_copy(xb, xs_hbm.at[i1], sm)

    for c in range(TOK_W // _CD):
        xb, i0, i1, sm = sets[c % 2]
        pltpu.make_async_copy(x_hbm.at[pl.ds(0, _CD), :], xb, sm).wait()
        pltpu.make_async_copy(x_hbm.at[pl.ds(0, _CD), :], xb, sm).wait()


@functools.partial(
    pl.kernel, mesh=_SC_MESH,
    out_type=jax.ShapeDtypeStruct((N, D), jnp.float32),
    scratch_types=[
        pltpu.VMEM((_CC, D), jnp.float32),
        pltpu.VMEM((_CC, D), jnp.float32),
        pltpu.VMEM((_CC, D), jnp.float32),
        pltpu.VMEM((_CC, D), jnp.float32),
        pltpu.VMEM((_CC,), jnp.int32),
        pltpu.VMEM((_CC,), jnp.int32),
        pltpu.VMEM((_CC,), jnp.int32),
        pltpu.VMEM((_CC,), jnp.int32),
        pltpu.VMEM((_CC, L), jnp.float32),
        pltpu.VMEM((_CC, L), jnp.float32),
        pltpu.VMEM((_CC, L), jnp.float32),
        pltpu.VMEM((_CC, L), jnp.float32),
        pltpu.SemaphoreType.DMA,
        pltpu.SemaphoreType.DMA,
    ],
)
def _combine(y_hbm, d0_hbm, d1_hbm, p0_hbm, p1_hbm, out_hbm,
             y0a, y1a, y0b, y1b, i0a, i1a, i0b, i1b,
             pb0a, pb1a, pb0b, pb1b, sema, semb):
    wid = jax.lax.axis_index("s") * NC + jax.lax.axis_index("c")
    sets = ((y0a, y1a, i0a, i1a, pb0a, pb1a, sema),
            (y0b, y1b, i0b, i1b, pb0b, pb1b, semb))
    ng = TOK_W // _CC // 2

    def fire(cidx, si):
        y0s, y1s, i0s, i1s, p0s, p1s, sm = sets[si]
        base = wid * TOK_W + cidx * _CC
        pltpu.sync_copy(d0_hbm.at[pl.ds(base, _CC)], i0s)
        pltpu.sync_copy(d1_hbm.at[pl.ds(base, _CC)], i1s)
        pltpu.sync_copy(p0_hbm.at[pl.ds(base, _CC), :], p0s)
        pltpu.sync_copy(p1_hbm.at[pl.ds(base, _CC), :], p1s)
        pltpu.async_copy(y_hbm.at[i0s], y0s, sm)
        pltpu.async_copy(y_hbm.at[i1s], y1s, sm)

    def drain(si):
        y0s, y1s, i0s, i1s, p0s, p1s, sm = sets[si]
        pltpu.make_async_copy(y_hbm.at[pl.ds(0, _CC), :], y0s, sm).wait()
        pltpu.make_async_copy(y_hbm.at[pl.ds(0, _CC), :], y1s, sm).wait()

    def compute_store(cidx, si):
        y0s, y1s, i0s, i1s, p0s, p1s, sm = sets[si]
        base = wid * TOK_W + cidx * _CC

        def row(r, rc):
            p0 = p0s[r, :]
            p1 = p1s[r, :]
            for cc in range(D // L):
                sl = pl.ds(cc * L, L)
                y0s[r, sl] = y0s[r, sl] * p0 + y1s[r, sl] * p1
            return rc

        jax.lax.fori_loop(0, _CC, row, 0)
        pltpu.sync_copy(y0s, out_hbm.at[pl.ds(base, _CC), :])

    fire(0, 0)
    fire(1, 1)

    def g_body(g, carry):
        drain(0)
        compute_store(2 * g, 0)

        @pl.when(g < ng - 1)
        def _():
            fire(2 * g + 2, 0)

        drain(1)
        compute_store(2 * g + 1, 1)

        @pl.when(g < ng - 1)
        def _():
            fire(2 * g + 3, 1)

        return carry

    jax.lax.fori_loop(0, ng, g_body, 0)


def kernel(x, Wr, fc1_w, fc1_b, fc2_w, fc2_b):
    xf = x.reshape(N, D)
    pr0, pr1, idx, xbf = _router(xf, Wr)
    dest, te = _meta(idx)
    d0 = dest[:, 0]
    d1 = dest[:, 1]
    x_sorted = _dispatch(xbf, d0, d1)
    y = _grouped_mlp(x_sorted, fc1_w, fc1_b, fc2_w, fc2_b, te[0])
    out = _combine(y, d0, d1, pr0, pr1)
    return out.reshape(B, S, D)


# speedup vs baseline: 1.0089x; 1.0089x over previous
import functools

import jax
import jax.numpy as jnp
from jax.experimental import pallas as pl
from jax.experimental.pallas import tpu as pltpu
from jax.experimental.pallas import tpu_sc as plsc

B, S, D, M, E, K = 4, 2048, 768, 3072, 8, 2
N = B * S
T = 512
NT = 40
NPAD = NT * T
NC, NS, L = 2, 16, 16
LB = 2 * L
NW = NC * NS
TOK_W = N // NW


def _rnd_bf16(v):
    lsb = jax.lax.shift_right_logical(v, 16) & 1
    return jax.lax.shift_right_logical(v + 0x7FFF + lsb, 16)


def _pack_bf16(yf):
    u = jax.lax.bitcast_convert_type(yf, jnp.int32)
    lo = _rnd_bf16(u[:, :D // 2])
    hi = _rnd_bf16(u[:, D // 2:])
    return lo | jax.lax.shift_left(hi, 16)


def _unpack_bf16(ui):
    lo = jax.lax.bitcast_convert_type(jax.lax.shift_left(ui, 16), jnp.float32)
    hi16 = jax.lax.shift_left(jax.lax.shift_right_logical(ui, 16), 16)
    hi = jax.lax.bitcast_convert_type(hi16, jnp.float32)
    return jnp.concatenate([lo, hi], axis=1)


def _cumsum0(a, n):
    k = 1
    while k < n:
        z = jnp.zeros((k,) + a.shape[1:], a.dtype)
        a = a + jnp.concatenate([z, a[:-k]], axis=0)
        k *= 2
    return a


_RB = 2048


def _router_body(x_ref, wr_ref, pr0_ref, pr1_ref, idx_ref, xbf_ref):
    x = x_ref[...]
    wr = wr_ref[...]
    xbf_ref[...] = _pack_bf16(x)
    logits = jax.lax.dot_general(x, wr, (((1,), (1,)), ((), ())),
                                 preferred_element_type=jnp.float32)
    m = jnp.max(logits, axis=-1, keepdims=True)
    p = jnp.exp(logits - m)
    probs = p / jnp.sum(p, axis=-1, keepdims=True)
    iota = jax.lax.broadcasted_iota(jnp.int32, (_RB, E), 1)
    m1 = jnp.max(probs, axis=-1, keepdims=True)
    i1 = jnp.min(jnp.where(probs == m1, iota, E), axis=-1, keepdims=True)
    probs2 = jnp.where(iota == i1, -1.0, probs)
    m2 = jnp.max(probs2, axis=-1, keepdims=True)
    i2 = jnp.min(jnp.where(probs2 == m2, iota, E), axis=-1, keepdims=True)
    ps = m1 + m2
    ones = jnp.ones((1, L), jnp.float32)
    pr0_ref[...] = (m1 / ps) * ones
    pr1_ref[...] = (m2 / ps) * ones
    idx_ref[...] = jnp.concatenate([i1, i2], axis=1)


def _router(xf, Wr):
    return pl.pallas_call(
        _router_body,
        grid=(N // _RB,),
        in_specs=[
            pl.BlockSpec((_RB, D), lambda i: (i, 0)),
            pl.BlockSpec((E, D), lambda i: (0, 0)),
        ],
        out_specs=(
            pl.BlockSpec((_RB, L), lambda i: (i, 0)),
            pl.BlockSpec((_RB, L), lambda i: (i, 0)),
            pl.BlockSpec((_RB, K), lambda i: (i, 0)),
            pl.BlockSpec((_RB, D // 2), lambda i: (i, 0)),
        ),
        out_shape=(
            jax.ShapeDtypeStruct((N, L), jnp.float32),
            jax.ShapeDtypeStruct((N, L), jnp.float32),
            jax.ShapeDtypeStruct((N, K), jnp.int32),
            jax.ShapeDtypeStruct((N, D // 2), jnp.int32),
        ),
    )(xf, Wr)


def _meta_body(idx_ref, dest_ref, te_ref):
    idx = idx_ref[...]
    iota = jax.lax.broadcasted_iota(jnp.int32, (N, E), 1)
    oh0 = (iota == idx[:, 0:1]).astype(jnp.float32)
    oh1 = (iota == idx[:, 1:2]).astype(jnp.float32)
    both = oh0 + oh1
    cum = _cumsum0(both, N)
    counts = cum[N - 1:N, :]
    excl = cum - both
    tiles_e = jnp.floor((counts + (T - 1)) * (1.0 / T))
    cumt = tiles_e
    k = 1
    while k < E:
        z = jnp.zeros((1, k), jnp.float32)
        cumt = cumt + jnp.concatenate([z, cumt[:, :-k]], axis=1)
        k *= 2
    off_pad = (cumt - tiles_e) * T
    base = off_pad + excl
    d0 = jnp.sum(base * oh0, axis=-1, keepdims=True)
    d1 = jnp.sum(base * oh1, axis=-1, keepdims=True)
    dest_ref[...] = jnp.concatenate([d0, d1], axis=1).astype(jnp.int32)

    j = jax.lax.broadcasted_iota(jnp.int32, (128, E), 0).astype(jnp.float32)
    te = jnp.sum((j >= cumt).astype(jnp.float32), axis=-1, keepdims=True)
    te = jnp.minimum(te, E - 1).astype(jnp.int32)
    te_ref[...] = te.reshape(1, 128)


def _meta(idx):
    return pl.pallas_call(
        _meta_body,
        out_shape=(
            jax.ShapeDtypeStruct((N, K), jnp.int32),
            jax.ShapeDtypeStruct((1, 128), jnp.int32),
        ),
    )(idx)


def _mlp_body(te_ref, x_ref, w1_ref, w2_ref, b1_ref, b2_ref, o_ref):
    i = pl.program_id(0)
    e = te_ref[i]
    xb = _unpack_bf16(x_ref[...])
    w1 = w1_ref[0]
    h = jax.lax.dot_general(xb, w1, (((1,), (1,)), ((), ())),
                            preferred_element_type=jnp.float32)
    h = h + b1_ref[pl.ds(e, 1), :]
    h = 0.5 * h * (1.0 + jax.lax.erf(h * 0.7071067811865476))
    w2 = w2_ref[0]
    y = jax.lax.dot_general(h, w2, (((1,), (1,)), ((), ())),
                            preferred_element_type=jnp.float32)
    o_ref[...] = y + b2_ref[pl.ds(e, 1), :]


def _grouped_mlp(x_sorted, fc1_w, fc1_b, fc2_w, fc2_b, te):
    grid_spec = pltpu.PrefetchScalarGridSpec(
        num_scalar_prefetch=1,
        grid=(NT,),
        in_specs=[
            pl.BlockSpec((T, D // 2), lambda i, te_ref: (i, 0)),
            pl.BlockSpec((1, M, D), lambda i, te_ref: (te_ref[i], 0, 0)),
            pl.BlockSpec((1, D, M), lambda i, te_ref: (te_ref[i], 0, 0)),
            pl.BlockSpec((E, M), lambda i, te_ref: (0, 0)),
            pl.BlockSpec((E, D), lambda i, te_ref: (0, 0)),
        ],
        out_specs=pl.BlockSpec((T, D), lambda i, te_ref: (i, 0)),
    )
    return pl.pallas_call(
        _mlp_body,
        grid_spec=grid_spec,
        out_shape=jax.ShapeDtypeStruct((NPAD, D), jnp.float32),
    )(te, x_sorted, fc1_w, fc2_w, fc1_b, fc2_b)


_SC_MESH = plsc.VectorSubcoreMesh(core_axis_name="c", subcore_axis_name="s")
_CD = 128
_CC = 32


@functools.partial(
    pl.kernel, mesh=_SC_MESH,
    out_type=jax.ShapeDtypeStruct((NPAD, D // 2), jnp.int32),
    scratch_types=[
        pltpu.VMEM((_CD, D // 2), jnp.int32),
        pltpu.VMEM((_CD, D // 2), jnp.int32),
        pltpu.VMEM((_CD,), jnp.int32),
        pltpu.VMEM((_CD,), jnp.int32),
        pltpu.VMEM((_CD,), jnp.int32),
        pltpu.VMEM((_CD,), jnp.int32),
        pltpu.SemaphoreType.DMA,
        pltpu.SemaphoreType.DMA,
    ],
)
def _dispatch(x_hbm, d0_hbm, d1_hbm, xs_hbm,
              xba, xbb, i0a, i1a, i0b, i1b, sema, semb):
    wid = jax.lax.axis_index("s") * NC + jax.lax.axis_index("c")
    sets = ((xba, i0a, i1a, sema), (xbb, i0b, i1b, semb))

    for c in range(TOK_W // _CD):
        xb, i0, i1, sm = sets[c % 2]
        base = wid * TOK_W + c * _CD
        pltpu.sync_copy(x_hbm.at[pl.ds(base, _CD), :], xb)
        pltpu.sync_copy(d0_hbm.at[pl.ds(base, _CD)], i0)
        pltpu.sync_copy(d1_hbm.at[pl.ds(base, _CD)], i1)
        pltpu.async_copy(xb, xs_hbm.at[i0], sm)
        pltpu.async_copy(xb, xs_hbm.at[i1], sm)

    for c in range(TOK_W // _CD):
        xb, i0, i1, sm = sets[c % 2]
        pltpu.make_async_copy(x_hbm.at[pl.ds(0, _CD), :], xb, sm).wait()
        pltpu.make_async_copy(x_hbm.at[pl.ds(0, _CD), :], xb, sm).wait()


@functools.partial(
    pl.kernel, mesh=_SC_MESH,
    out_type=jax.ShapeDtypeStruct((N, D), jnp.float32),
    scratch_types=[
        pltpu.VMEM((_CC, D), jnp.float32),
        pltpu.VMEM((_CC, D), jnp.float32),
        pltpu.VMEM((_CC, D), jnp.float32),
        pltpu.VMEM((_CC, D), jnp.float32),
        pltpu.VMEM((_CC,), jnp.int32),
        pltpu.VMEM((_CC,), jnp.int32),
        pltpu.VMEM((_CC,), jnp.int32),
        pltpu.VMEM((_CC,), jnp.int32),
        pltpu.VMEM((_CC, L), jnp.float32),
        pltpu.VMEM((_CC, L), jnp.float32),
        pltpu.VMEM((_CC, L), jnp.float32),
        pltpu.VMEM((_CC, L), jnp.float32),
        pltpu.SemaphoreType.DMA,
        pltpu.SemaphoreType.DMA,
    ],
)
def _combine(y_hbm, d0_hbm, d1_hbm, p0_hbm, p1_hbm, out_hbm,
             y0a, y1a, y0b, y1b, i0a, i1a, i0b, i1b,
             pb0a, pb1a, pb0b, pb1b, sema, semb):
    wid = jax.lax.axis_index("s") * NC + jax.lax.axis_index("c")
    sets = ((y0a, y1a, i0a, i1a, pb0a, pb1a, sema),
            (y0b, y1b, i0b, i1b, pb0b, pb1b, semb))
    ng = TOK_W // _CC // 2

    def fire(cidx, si):
        y0s, y1s, i0s, i1s, p0s, p1s, sm = sets[si]
        base = wid * TOK_W + cidx * _CC
        pltpu.sync_copy(d0_hbm.at[pl.ds(base, _CC)], i0s)
        pltpu.sync_copy(d1_hbm.at[pl.ds(base, _CC)], i1s)
        pltpu.sync_copy(p0_hbm.at[pl.ds(base, _CC), :], p0s)
        pltpu.sync_copy(p1_hbm.at[pl.ds(base, _CC), :], p1s)
        pltpu.async_copy(y_hbm.at[i0s], y0s, sm)
        pltpu.async_copy(y_hbm.at[i1s], y1s, sm)

    def drain(si):
        y0s, y1s, i0s, i1s, p0s, p1s, sm = sets[si]
        pltpu.make_async_copy(y_hbm.at[pl.ds(0, _CC), :], y0s, sm).wait()
        pltpu.make_async_copy(y_hbm.at[pl.ds(0, _CC), :], y1s, sm).wait()

    def compute_store(cidx, si):
        y0s, y1s, i0s, i1s, p0s, p1s, sm = sets[si]
        base = wid * TOK_W + cidx * _CC

        def row(r, rc):
            p0 = p0s[r, :]
            p1 = p1s[r, :]
            for cc in range(D // L):
                sl = pl.ds(cc * L, L)
                y0s[r, sl] = y0s[r, sl] * p0 + y1s[r, sl] * p1
            return rc

        jax.lax.fori_loop(0, _CC, row, 0)
        pltpu.sync_copy(y0s, out_hbm.at[pl.ds(base, _CC), :])

    fire(0, 0)
    fire(1, 1)

    def g_body(g, carry):
        drain(0)
        compute_store(2 * g, 0)

        @pl.when(g < ng - 1)
        def _():
            fire(2 * g + 2, 0)

        drain(1)
        compute_store(2 * g + 1, 1)

        @pl.when(g < ng - 1)
        def _():
            fire(2 * g + 3, 1)

        return carry

    jax.lax.fori_loop(0, ng, g_body, 0)


def kernel(x, Wr, fc1_w, fc1_b, fc2_w, fc2_b):
    xf = x.reshape(N, D)
    pr0, pr1, idx, xbf = _router(xf, Wr)
    dest, te = _meta(idx)
    d0 = dest[:, 0]
    d1 = dest[:, 1]
    x_sorted = _dispatch(xbf, d0, d1)
    y = _grouped_mlp(x_sorted, fc1_w, fc1_b, fc2_w, fc2_b, te[0])
    out = _combine(y, d0, d1, pr0, pr1)
    return out.reshape(B, S, D)


# speedup vs baseline: 1.0210x; 1.0120x over previous
import functools

import jax
import jax.numpy as jnp
from jax.experimental import pallas as pl
from jax.experimental.pallas import tpu as pltpu
from jax.experimental.pallas import tpu_sc as plsc

B, S, D, M, E, K = 4, 2048, 768, 3072, 8, 2
N = B * S
T = 512
NT = 39
NPAD = NT * T
NC, NS, L = 2, 16, 16
LB = 2 * L
NW = NC * NS
TOK_W = N // NW


def _rnd_bf16(v):
    lsb = jax.lax.shift_right_logical(v, 16) & 1
    return jax.lax.shift_right_logical(v + 0x7FFF + lsb, 16)


def _pack_bf16(yf):
    u = jax.lax.bitcast_convert_type(yf, jnp.int32)
    lo = _rnd_bf16(u[:, :D // 2])
    hi = _rnd_bf16(u[:, D // 2:])
    return lo | jax.lax.shift_left(hi, 16)


def _unpack_bf16(ui):
    lo = jax.lax.bitcast_convert_type(jax.lax.shift_left(ui, 16), jnp.float32)
    hi16 = jax.lax.shift_left(jax.lax.shift_right_logical(ui, 16), 16)
    hi = jax.lax.bitcast_convert_type(hi16, jnp.float32)
    return jnp.concatenate([lo, hi], axis=1)


def _cumsum0(a, n):
    k = 1
    while k < n:
        z = jnp.zeros((k,) + a.shape[1:], a.dtype)
        a = a + jnp.concatenate([z, a[:-k]], axis=0)
        k *= 2
    return a


_RB = 2048


def _router_body(x_ref, wr_ref, pr0_ref, pr1_ref, idx_ref, xbf_ref):
    x = x_ref[...]
    wr = wr_ref[...]
    xbf_ref[...] = _pack_bf16(x)
    logits = jax.lax.dot_general(x, wr, (((1,), (1,)), ((), ())),
                                 preferred_element_type=jnp.float32)
    m = jnp.max(logits, axis=-1, keepdims=True)
    p = jnp.exp(logits - m)
    probs = p / jnp.sum(p, axis=-1, keepdims=True)
    iota = jax.lax.broadcasted_iota(jnp.int32, (_RB, E), 1)
    m1 = jnp.max(probs, axis=-1, keepdims=True)
    i1 = jnp.min(jnp.where(probs == m1, iota, E), axis=-1, keepdims=True)
    probs2 = jnp.where(iota == i1, -1.0, probs)
    m2 = jnp.max(probs2, axis=-1, keepdims=True)
    i2 = jnp.min(jnp.where(probs2 == m2, iota, E), axis=-1, keepdims=True)
    ps = m1 + m2
    ones = jnp.ones((1, L), jnp.float32)
    pr0_ref[...] = (m1 / ps) * ones
    pr1_ref[...] = (m2 / ps) * ones
    idx_ref[...] = jnp.concatenate([i1, i2], axis=1)


def _router(xf, Wr):
    return pl.pallas_call(
        _router_body,
        grid=(N // _RB,),
        in_specs=[
            pl.BlockSpec((_RB, D), lambda i: (i, 0)),
            pl.BlockSpec((E, D), lambda i: (0, 0)),
        ],
        out_specs=(
            pl.BlockSpec((_RB, L), lambda i: (i, 0)),
            pl.BlockSpec((_RB, L), lambda i: (i, 0)),
            pl.BlockSpec((_RB, K), lambda i: (i, 0)),
            pl.BlockSpec((_RB, D // 2), lambda i: (i, 0)),
        ),
        out_shape=(
            jax.ShapeDtypeStruct((N, L), jnp.float32),
            jax.ShapeDtypeStruct((N, L), jnp.float32),
            jax.ShapeDtypeStruct((N, K), jnp.int32),
            jax.ShapeDtypeStruct((N, D // 2), jnp.int32),
        ),
    )(xf, Wr)


def _meta_body(idx_ref, dest_ref, te_ref):
    idx = idx_ref[...]
    iota = jax.lax.broadcasted_iota(jnp.int32, (N, E), 1)
    oh0 = (iota == idx[:, 0:1]).astype(jnp.float32)
    oh1 = (iota == idx[:, 1:2]).astype(jnp.float32)
    both = oh0 + oh1
    cum = _cumsum0(both, N)
    counts = cum[N - 1:N, :]
    excl = cum - both
    tiles_e = jnp.floor((counts + (T - 1)) * (1.0 / T))
    cumt = tiles_e
    k = 1
    while k < E:
        z = jnp.zeros((1, k), jnp.float32)
        cumt = cumt + jnp.concatenate([z, cumt[:, :-k]], axis=1)
        k *= 2
    off_pad = (cumt - tiles_e) * T
    base = off_pad + excl
    d0 = jnp.sum(base * oh0, axis=-1, keepdims=True)
    d1 = jnp.sum(base * oh1, axis=-1, keepdims=True)
    dest_ref[...] = jnp.concatenate([d0, d1], axis=1).astype(jnp.int32)

    j = jax.lax.broadcasted_iota(jnp.int32, (128, E), 0).astype(jnp.float32)
    te = jnp.sum((j >= cumt).astype(jnp.float32), axis=-1, keepdims=True)
    te = jnp.minimum(te, E - 1).astype(jnp.int32)
    te_ref[...] = te.reshape(1, 128)


def _meta(idx):
    return pl.pallas_call(
        _meta_body,
        out_shape=(
            jax.ShapeDtypeStruct((N, K), jnp.int32),
            jax.ShapeDtypeStruct((1, 128), jnp.int32),
        ),
    )(idx)


def _mlp_body(te_ref, x_ref, w1_ref, w2_ref, b1_ref, b2_ref, o_ref):
    i = pl.program_id(0)
    e = te_ref[i]
    xb = _unpack_bf16(x_ref[...])
    w1 = w1_ref[0]
    h = jax.lax.dot_general(xb, w1, (((1,), (1,)), ((), ())),
                            preferred_element_type=jnp.float32)
    h = h + b1_ref[pl.ds(e, 1), :]
    h = 0.5 * h * (1.0 + jax.lax.erf(h * 0.7071067811865476))
    w2 = w2_ref[0]
    y = jax.lax.dot_general(h, w2, (((1,), (1,)), ((), ())),
                            preferred_element_type=jnp.float32)
    o_ref[...] = y + b2_ref[pl.ds(e, 1), :]


def _grouped_mlp(x_sorted, fc1_w, fc1_b, fc2_w, fc2_b, te):
    grid_spec = pltpu.PrefetchScalarGridSpec(
        num_scalar_prefetch=1,
        grid=(NT,),
        in_specs=[
            pl.BlockSpec((T, D // 2), lambda i, te_ref: (i, 0)),
            pl.BlockSpec((1, M, D), lambda i, te_ref: (te_ref[i], 0, 0)),
            pl.BlockSpec((1, D, M), lambda i, te_ref: (te_ref[i], 0, 0)),
            pl.BlockSpec((E, M), lambda i, te_ref: (0, 0)),
            pl.BlockSpec((E, D), lambda i, te_ref: (0, 0)),
        ],
        out_specs=pl.BlockSpec((T, D), lambda i, te_ref: (i, 0)),
    )
    return pl.pallas_call(
        _mlp_body,
        grid_spec=grid_spec,
        out_shape=jax.ShapeDtypeStruct((NPAD, D), jnp.float32),
    )(te, x_sorted, fc1_w, fc2_w, fc1_b, fc2_b)


_SC_MESH = plsc.VectorSubcoreMesh(core_axis_name="c", subcore_axis_name="s")
_CD = 128
_CC = 32


@functools.partial(
    pl.kernel, mesh=_SC_MESH,
    out_type=jax.ShapeDtypeStruct((NPAD, D // 2), jnp.int32),
    scratch_types=[
        pltpu.VMEM((_CD, D // 2), jnp.int32),
        pltpu.VMEM((_CD, D // 2), jnp.int32),
        pltpu.VMEM((_CD,), jnp.int32),
        pltpu.VMEM((_CD,), jnp.int32),
        pltpu.VMEM((_CD,), jnp.int32),
        pltpu.VMEM((_CD,), jnp.int32),
        pltpu.SemaphoreType.DMA,
        pltpu.SemaphoreType.DMA,
    ],
)
def _dispatch(x_hbm, d0_hbm, d1_hbm, xs_hbm,
              xba, xbb, i0a, i1a, i0b, i1b, sema, semb):
    wid = jax.lax.axis_index("s") * NC + jax.lax.axis_index("c")
    sets = ((xba, i0a, i1a, sema), (xbb, i0b, i1b, semb))

    for c in range(TOK_W // _CD):
        xb, i0, i1, sm = sets[c % 2]
        base = wid * TOK_W + c * _CD
        pltpu.sync_copy(x_hbm.at[pl.ds(base, _CD), :], xb)
        pltpu.sync_copy(d0_hbm.at[pl.ds(base, _CD)], i0)
        pltpu.sync_copy(d1_hbm.at[pl.ds(base, _CD)], i1)
        pltpu.async_copy(xb, xs_hbm.at[i0], sm)
        pltpu.async_copy(xb, xs_hbm.at[i1], sm)

    for c in range(TOK_W // _CD):
        xb, i0, i1, sm = sets[c % 2]
        pltpu.make_async_copy(x_hbm.at[pl.ds(0, _CD), :], xb, sm).wait()
        pltpu.make_async_copy(x_hbm.at[pl.ds(0, _CD), :], xb, sm).wait()


@functools.partial(
    pl.kernel, mesh=_SC_MESH,
    out_type=jax.ShapeDtypeStruct((N, D), jnp.float32),
    scratch_types=[
        pltpu.VMEM((_CC, D), jnp.float32),
        pltpu.VMEM((_CC, D), jnp.float32),
        pltpu.VMEM((_CC, D), jnp.float32),
        pltpu.VMEM((_CC, D), jnp.float32),
        pltpu.VMEM((_CC,), jnp.int32),
        pltpu.VMEM((_CC,), jnp.int32),
        pltpu.VMEM((_CC,), jnp.int32),
        pltpu.VMEM((_CC,), jnp.int32),
        pltpu.VMEM((_CC, L), jnp.float32),
        pltpu.VMEM((_CC, L), jnp.float32),
        pltpu.VMEM((_CC, L), jnp.float32),
        pltpu.VMEM((_CC, L), jnp.float32),
        pltpu.SemaphoreType.DMA,
        pltpu.SemaphoreType.DMA,
    ],
)
def _combine(y_hbm, d0_hbm, d1_hbm, p0_hbm, p1_hbm, out_hbm,
             y0a, y1a, y0b, y1b, i0a, i1a, i0b, i1b,
             pb0a, pb1a, pb0b, pb1b, sema, semb):
    wid = jax.lax.axis_index("s") * NC + jax.lax.axis_index("c")
    sets = ((y0a, y1a, i0a, i1a, pb0a, pb1a, sema),
            (y0b, y1b, i0b, i1b, pb0b, pb1b, semb))
    ng = TOK_W // _CC // 2

    def fire(cidx, si):
        y0s, y1s, i0s, i1s, p0s, p1s, sm = sets[si]
        base = wid * TOK_W + cidx * _CC
        pltpu.sync_copy(d0_hbm.at[pl.ds(base, _CC)], i0s)
        pltpu.sync_copy(d1_hbm.at[pl.ds(base, _CC)], i1s)
        pltpu.sync_copy(p0_hbm.at[pl.ds(base, _CC), :], p0s)
        pltpu.sync_copy(p1_hbm.at[pl.ds(base, _CC), :], p1s)
        pltpu.async_copy(y_hbm.at[i0s], y0s, sm)
        pltpu.async_copy(y_hbm.at[i1s], y1s, sm)

    def drain(si):
        y0s, y1s, i0s, i1s, p0s, p1s, sm = sets[si]
        pltpu.make_async_copy(y_hbm.at[pl.ds(0, _CC), :], y0s, sm).wait()
        pltpu.make_async_copy(y_hbm.at[pl.ds(0, _CC), :], y1s, sm).wait()

    def compute_store(cidx, si):
        y0s, y1s, i0s, i1s, p0s, p1s, sm = sets[si]
        base = wid * TOK_W + cidx * _CC

        def row(r, rc):
            p0 = p0s[r, :]
            p1 = p1s[r, :]
            for cc in range(D // L):
                sl = pl.ds(cc * L, L)
                y0s[r, sl] = y0s[r, sl] * p0 + y1s[r, sl] * p1
            return rc

        jax.lax.fori_loop(0, _CC, row, 0)
        pltpu.sync_copy(y0s, out_hbm.at[pl.ds(base, _CC), :])

    fire(0, 0)
    fire(1, 1)

    def g_body(g, carry):
        drain(0)
        compute_store(2 * g, 0)

        @pl.when(g < ng - 1)
        def _():
            fire(2 * g + 2, 0)

        drain(1)
        compute_store(2 * g + 1, 1)

        @pl.when(g < ng - 1)
        def _():
            fire(2 * g + 3, 1)

        return carry

    jax.lax.fori_loop(0, ng, g_body, 0)


def kernel(x, Wr, fc1_w, fc1_b, fc2_w, fc2_b):
    xf = x.reshape(N, D)
    pr0, pr1, idx, xbf = _router(xf, Wr)
    dest, te = _meta(idx)
    d0 = dest[:, 0]
    d1 = dest[:, 1]
    x_sorted = _dispatch(xbf, d0, d1)
    y = _grouped_mlp(x_sorted, fc1_w, fc1_b, fc2_w, fc2_b, te[0])
    out = _combine(y, d0, d1, pr0, pr1)
    return out.reshape(B, S, D)


# speedup vs baseline: 1.0226x; 1.0016x over previous
import functools

import jax
import jax.numpy as jnp
from jax.experimental import pallas as pl
from jax.experimental.pallas import tpu as pltpu
from jax.experimental.pallas import tpu_sc as plsc

B, S, D, M, E, K = 4, 2048, 768, 3072, 8, 2
N = B * S
T = 512
NT = 39
NPAD = NT * T
NC, NS, L = 2, 16, 16
NW = NC * NS
TOK_W = N // NW


def _rnd_bf16(v):
    lsb = jax.lax.shift_right_logical(v, 16) & 1
    return jax.lax.shift_right_logical(v + 0x7FFF + lsb, 16)


def _pack_bf16(yf):
    u = jax.lax.bitcast_convert_type(yf, jnp.int32)
    lo = _rnd_bf16(u[:, :D // 2])
    hi = _rnd_bf16(u[:, D // 2:])
    return lo | jax.lax.shift_left(hi, 16)


def _unpack_bf16(ui):
    lo = jax.lax.bitcast_convert_type(jax.lax.shift_left(ui, 16), jnp.float32)
    hi16 = jax.lax.shift_left(jax.lax.shift_right_logical(ui, 16), 16)
    hi = jax.lax.bitcast_convert_type(hi16, jnp.float32)
    return jnp.concatenate([lo, hi], axis=1)


def _cumsum0(a, n):
    k = 1
    while k < n:
        z = jnp.zeros((k,) + a.shape[1:], a.dtype)
        a = a + jnp.concatenate([z, a[:-k]], axis=0)
        k *= 2
    return a


_RB = 2048


def _router_body(x_ref, wr_ref, pr0_ref, pr1_ref, idx_ref, xbf_ref):
    x = x_ref[...]
    wr = wr_ref[...]
    xbf_ref[...] = _pack_bf16(x)
    logits = jax.lax.dot_general(x, wr, (((1,), (1,)), ((), ())),
                                 preferred_element_type=jnp.float32)
    m = jnp.max(logits, axis=-1, keepdims=True)
    p = jnp.exp(logits - m)
    probs = p / jnp.sum(p, axis=-1, keepdims=True)
    iota = jax.lax.broadcasted_iota(jnp.int32, (_RB, E), 1)
    m1 = jnp.max(probs, axis=-1, keepdims=True)
    i1 = jnp.min(jnp.where(probs == m1, iota, E), axis=-1, keepdims=True)
    probs2 = jnp.where(iota == i1, -1.0, probs)
    m2 = jnp.max(probs2, axis=-1, keepdims=True)
    i2 = jnp.min(jnp.where(probs2 == m2, iota, E), axis=-1, keepdims=True)
    ps = m1 + m2
    ones = jnp.ones((1, L), jnp.float32)
    pr0_ref[...] = (m1 / ps) * ones
    pr1_ref[...] = (m2 / ps) * ones
    idx_ref[...] = jnp.concatenate([i1, i2], axis=1)


def _router(xf, Wr):
    return pl.pallas_call(
        _router_body,
        grid=(N // _RB,),
        in_specs=[
            pl.BlockSpec((_RB, D), lambda i: (i, 0)),
            pl.BlockSpec((E, D), lambda i: (0, 0)),
        ],
        out_specs=(
            pl.BlockSpec((_RB, L), lambda i: (i, 0)),
            pl.BlockSpec((_RB, L), lambda i: (i, 0)),
            pl.BlockSpec((_RB, K), lambda i: (i, 0)),
            pl.BlockSpec((_RB, D // 2), lambda i: (i, 0)),
        ),
        out_shape=(
            jax.ShapeDtypeStruct((N, L), jnp.float32),
            jax.ShapeDtypeStruct((N, L), jnp.float32),
            jax.ShapeDtypeStruct((N, K), jnp.int32),
            jax.ShapeDtypeStruct((N, D // 2), jnp.int32),
        ),
    )(xf, Wr)


def _meta_body(idx_ref, dest_ref, te_ref):
    idx = idx_ref[...]
    iota = jax.lax.broadcasted_iota(jnp.int32, (N, E), 1)
    oh0 = (iota == idx[:, 0:1]).astype(jnp.float32)
    oh1 = (iota == idx[:, 1:2]).astype(jnp.float32)
    both = oh0 + oh1
    cum = _cumsum0(both, N)
    counts = cum[N - 1:N, :]
    excl = cum - both
    tiles_e = jnp.floor((counts + (T - 1)) * (1.0 / T))
    cumt = tiles_e
    k = 1
    while k < E:
        z = jnp.zeros((1, k), jnp.float32)
        cumt = cumt + jnp.concatenate([z, cumt[:, :-k]], axis=1)
        k *= 2
    off_pad = (cumt - tiles_e) * T
    base = off_pad + excl
    d0 = jnp.sum(base * oh0, axis=-1, keepdims=True)
    d1 = jnp.sum(base * oh1, axis=-1, keepdims=True)
    dest_ref[...] = jnp.concatenate([d0, d1], axis=1).astype(jnp.int32)

    j = jax.lax.broadcasted_iota(jnp.int32, (128, E), 0).astype(jnp.float32)
    te = jnp.sum((j >= cumt).astype(jnp.float32), axis=-1, keepdims=True)
    te = jnp.minimum(te, E - 1).astype(jnp.int32)
    te_ref[...] = te.reshape(1, 128)


def _meta(idx):
    return pl.pallas_call(
        _meta_body,
        out_shape=(
            jax.ShapeDtypeStruct((N, K), jnp.int32),
            jax.ShapeDtypeStruct((1, 128), jnp.int32),
        ),
    )(idx)


def _mlp_body(te_ref, x_ref, w1_ref, w2_ref, b1_ref, b2_ref, o_ref):
    i = pl.program_id(0)
    e = te_ref[i]
    xb = _unpack_bf16(x_ref[...])
    w1 = w1_ref[0]
    h = jax.lax.dot_general(xb, w1, (((1,), (1,)), ((), ())),
                            preferred_element_type=jnp.float32)
    h = h + b1_ref[pl.ds(e, 1), :]
    h = 0.5 * h * (1.0 + jax.lax.erf(h * 0.7071067811865476))
    w2 = w2_ref[0]
    y = jax.lax.dot_general(h, w2, (((1,), (1,)), ((), ())),
                            preferred_element_type=jnp.float32)
    o_ref[...] = y + b2_ref[pl.ds(e, 1), :]


def _grouped_mlp(x_sorted, fc1_w, fc1_b, fc2_w, fc2_b, te):
    grid_spec = pltpu.PrefetchScalarGridSpec(
        num_scalar_prefetch=1,
        grid=(NT,),
        in_specs=[
            pl.BlockSpec((T, D // 2), lambda i, te_ref: (i, 0)),
            pl.BlockSpec((1, M, D), lambda i, te_ref: (te_ref[i], 0, 0)),
            pl.BlockSpec((1, D, M), lambda i, te_ref: (te_ref[i], 0, 0)),
            pl.BlockSpec((E, M), lambda i, te_ref: (0, 0)),
            pl.BlockSpec((E, D), lambda i, te_ref: (0, 0)),
        ],
        out_specs=pl.BlockSpec((T, D), lambda i, te_ref: (i, 0)),
    )
    return pl.pallas_call(
        _mlp_body,
        grid_spec=grid_spec,
        out_shape=jax.ShapeDtypeStruct((NPAD, D), jnp.float32),
    )(te, x_sorted, fc1_w, fc2_w, fc1_b, fc2_b)


_SC_MESH = plsc.VectorSubcoreMesh(core_axis_name="c", subcore_axis_name="s")
_CD = 128
_CC = 32


@functools.partial(
    pl.kernel, mesh=_SC_MESH,
    out_type=jax.ShapeDtypeStruct((NPAD, D // 2), jnp.int32),
    scratch_types=[
        pltpu.VMEM((_CD, D // 2), jnp.int32),
        pltpu.VMEM((_CD, D // 2), jnp.int32),
        pltpu.VMEM((_CD,), jnp.int32),
        pltpu.VMEM((_CD,), jnp.int32),
        pltpu.VMEM((_CD,), jnp.int32),
        pltpu.VMEM((_CD,), jnp.int32),
        pltpu.SemaphoreType.DMA,
        pltpu.SemaphoreType.DMA,
    ],
)
def _dispatch(x_hbm, d0_hbm, d1_hbm, xs_hbm,
              xba, xbb, i0a, i1a, i0b, i1b, sema, semb):
    wid = jax.lax.axis_index("s") * NC + jax.lax.axis_index("c")
    sets = ((xba, i0a, i1a, sema), (xbb, i0b, i1b, semb))

    for c in range(TOK_W // _CD):
        xb, i0, i1, sm = sets[c % 2]
        base = wid * TOK_W + c * _CD
        pltpu.sync_copy(x_hbm.at[pl.ds(base, _CD), :], xb)
        pltpu.sync_copy(d0_hbm.at[pl.ds(base, _CD)], i0)
        pltpu.sync_copy(d1_hbm.at[pl.ds(base, _CD)], i1)
        pltpu.async_copy(xb, xs_hbm.at[i0], sm)
        pltpu.async_copy(xb, xs_hbm.at[i1], sm)

    for c in range(TOK_W // _CD):
        xb, i0, i1, sm = sets[c % 2]
        pltpu.make_async_copy(x_hbm.at[pl.ds(0, _CD), :], xb, sm).wait()
        pltpu.make_async_copy(x_hbm.at[pl.ds(0, _CD), :], xb, sm).wait()


@functools.partial(
    pl.kernel, mesh=_SC_MESH,
    out_type=jax.ShapeDtypeStruct((N, D), jnp.float32),
    scratch_types=[
        pltpu.VMEM((_CC, D), jnp.float32),
        pltpu.VMEM((_CC, D), jnp.float32),
        pltpu.VMEM((_CC, D), jnp.float32),
        pltpu.VMEM((_CC, D), jnp.float32),
        pltpu.VMEM((_CC,), jnp.int32),
        pltpu.VMEM((_CC,), jnp.int32),
        pltpu.VMEM((_CC,), jnp.int32),
        pltpu.VMEM((_CC,), jnp.int32),
        pltpu.VMEM((_CC, L), jnp.float32),
        pltpu.VMEM((_CC, L), jnp.float32),
        pltpu.VMEM((_CC, L), jnp.float32),
        pltpu.VMEM((_CC, L), jnp.float32),
        pltpu.SemaphoreType.DMA,
        pltpu.SemaphoreType.DMA,
    ],
)
def _combine(y_hbm, d0_hbm, d1_hbm, p0_hbm, p1_hbm, out_hbm,
             y0a, y1a, y0b, y1b, i0a, i1a, i0b, i1b,
             pb0a, pb1a, pb0b, pb1b, sema, semb):
    wid = jax.lax.axis_index("s") * NC + jax.lax.axis_index("c")
    sets = ((y0a, y1a, i0a, i1a, pb0a, pb1a, sema),
            (y0b, y1b, i0b, i1b, pb0b, pb1b, semb))
    ng = TOK_W // _CC // 2

    def fire(cidx, si):
        y0s, y1s, i0s, i1s, p0s, p1s, sm = sets[si]
        base = wid * TOK_W + cidx * _CC
        pltpu.sync_copy(d0_hbm.at[pl.ds(base, _CC)], i0s)
        pltpu.sync_copy(d1_hbm.at[pl.ds(base, _CC)], i1s)
        pltpu.sync_copy(p0_hbm.at[pl.ds(base, _CC), :], p0s)
        pltpu.sync_copy(p1_hbm.at[pl.ds(base, _CC), :], p1s)
        pltpu.async_copy(y_hbm.at[i0s], y0s, sm)
        pltpu.async_copy(y_hbm.at[i1s], y1s, sm)

    def drain(si):
        y0s, y1s, i0s, i1s, p0s, p1s, sm = sets[si]
        pltpu.make_async_copy(y_hbm.at[pl.ds(0, _CC), :], y0s, sm).wait()
        pltpu.make_async_copy(y_hbm.at[pl.ds(0, _CC), :], y1s, sm).wait()

    def compute_store(cidx, si):
        y0s, y1s, i0s, i1s, p0s, p1s, sm = sets[si]
        base = wid * TOK_W + cidx * _CC

        def row(r, rc):
            p0 = p0s[r, :]
            p1 = p1s[r, :]
            for cc in range(D // L):
                sl = pl.ds(cc * L, L)
                y0s[r, sl] = y0s[r, sl] * p0 + y1s[r, sl] * p1
            return rc

        jax.lax.fori_loop(0, _CC, row, 0)
        pltpu.sync_copy(y0s, out_hbm.at[pl.ds(base, _CC), :])

    fire(0, 0)
    fire(1, 1)

    def g_body(g, carry):
        drain(0)
        compute_store(2 * g, 0)

        @pl.when(g < ng - 1)
        def _():
            fire(2 * g + 2, 0)

        drain(1)
        compute_store(2 * g + 1, 1)

        @pl.when(g < ng - 1)
        def _():
            fire(2 * g + 3, 1)

        return carry

    jax.lax.fori_loop(0, ng, g_body, 0)


def kernel(x, Wr, fc1_w, fc1_b, fc2_w, fc2_b):
    xf = x.reshape(N, D)
    pr0, pr1, idx, xbf = _router(xf, Wr)
    dest, te = _meta(idx)
    d0 = dest[:, 0]
    d1 = dest[:, 1]
    x_sorted = _dispatch(xbf, d0, d1)
    y = _grouped_mlp(x_sorted, fc1_w, fc1_b, fc2_w, fc2_b, te[0])
    out = _combine(y, d0, d1, pr0, pr1)
    return out.reshape(B, S, D)
